# baseline (speedup 1.0000x reference)
import jax
import jax.numpy as jnp
from jax.experimental import pallas as pl


def _bn(x, g, b, eps=1e-5):
    mu = jnp.mean(x, axis=0)
    var = jnp.var(x, axis=0)
    return g * (x - mu) / jnp.sqrt(var + eps) + b


def _mlp(x, layers):
    for (W, b, g, be) in layers:
        x = jax.nn.relu(x @ W.T + b)
        x = _bn(x, g, be)
    return x


def _knn(x, batch, k):
    sq = jnp.sum(x * x, axis=1)
    d = sq[:, None] + sq[None, :] - 2.0 * (x @ x.T)
    d = jnp.where(batch[:, None] != batch[None, :], jnp.inf, d)
    _, idx = jax.lax.top_k(-d, k)
    return idx


def _edge_conv(x, batch, layers, k):
    idx = _knn(x, batch, k)
    xj = x[idx]
    xi = jnp.broadcast_to(x[:, None, :], xj.shape)
    e = jnp.concatenate([xi, xj - xi], axis=-1)
    m = _mlp(e.reshape(-1, e.shape[-1]), layers)
    m = m.reshape(x.shape[0], k, -1)
    return jnp.max(m, axis=1)


def _final_kernel(x_ref, w_ref, b_ref, o_ref):
    z = jnp.dot(x_ref[...], w_ref[...], preferred_element_type=jnp.float32)
    z = z + b_ref[...]
    mx = jnp.max(z, axis=1, keepdims=True)
    ls = jnp.log(jnp.sum(jnp.exp(z - mx), axis=1, keepdims=True)) + mx
    o_ref[...] = z - ls


def kernel(pos, batch, params):
    K = 30
    x1 = _edge_conv(pos, batch, params['conv1'], K)
    x2 = _edge_conv(x1, batch, params['conv2'], K)
    x3 = _edge_conv(x2, batch, params['conv3'], K)
    out = _mlp(jnp.concatenate([x1, x2, x3], axis=1), params['lin1'])
    out = _mlp(out, params['head1'])
    out = _mlp(out, params['head2'])
    W, b = params['final']
    N = out.shape[0]
    o = pl.pallas_call(
        _final_kernel,
        out_shape=jax.ShapeDtypeStruct((N, 13), jnp.float32),
        grid=(N // 512,),
        in_specs=[
            pl.BlockSpec((512, 128), lambda i: (i, 0)),
            pl.BlockSpec((128, 13), lambda i: (0, 0)),
            pl.BlockSpec((1, 13), lambda i: (0, 0)),
        ],
        out_specs=pl.BlockSpec((512, 13), lambda i: (i, 0)),
    )(out, W.T, b[None, :])
    return o


# speedup vs baseline: 5.9069x; 5.9069x over previous
import functools

import jax
import jax.numpy as jnp
from jax import lax
from jax.experimental import pallas as pl
from jax.experimental.pallas import tpu as pltpu
from jax.experimental.pallas import tpu_sc as plsc

N = 8192
KNN = 30
NG = 8
RB = 128
CT = 512
NBLK = N // RB
RB2 = 256
NB2 = N // RB2
EPS = 1e-5
F32 = jnp.float32
BF16 = jnp.bfloat16


def _knn_body(t0_ref, t1_ref, xn_ref, sqr_ref, sqc_ref, batr_ref, batc_ref,
              idx_ref, d_scr):
    i = pl.program_id(0)
    a0 = t0_ref[i]
    a1 = t1_ref[i]
    xb = xn_ref[pl.ds(i * RB, RB), :].astype(BF16)
    sqr = sqr_ref[...]
    batr = batr_ref[...]

    def fill(t, _):
        c0 = t * CT
        xt = xn_ref[pl.ds(c0, CT), :].astype(BF16)
        d = lax.dot_general(xb, xt, (((1,), (1,)), ((), ())),
                            preferred_element_type=F32)
        d = sqr + sqc_ref[:, pl.ds(c0, CT)] - 2.0 * d
        bc = batc_ref[:, pl.ds(c0, CT)]
        d = jnp.where(batr != bc, jnp.inf, d)
        d_scr[:, pl.ds(c0, CT)] = d
        return 0

    lax.fori_loop(a0, a1, fill, 0)

    lane_t = lax.broadcasted_iota(jnp.int32, (RB, 32), 1)

    def extract(t, carry):
        prevj, acc = carry

        def scan_tile(tt, c2):
            mval, marg = c2
            c0 = tt * CT
            d = d_scr[:, pl.ds(c0, CT)]
            ci = lax.broadcasted_iota(jnp.int32, (RB, CT), 1) + c0
            d = jnp.where(ci == prevj, jnp.inf, d)
            d_scr[:, pl.ds(c0, CT)] = d
            tmin = jnp.min(d, axis=1, keepdims=True)
            targ = jnp.min(jnp.where(d <= tmin, ci, jnp.int32(2**30)),
                           axis=1, keepdims=True)
            upd = tmin < mval
            return (jnp.where(upd, tmin, mval), jnp.where(upd, targ, marg))

        mval0 = jnp.full((RB, 1), jnp.inf, F32)
        marg0 = jnp.zeros((RB, 1), jnp.int32)
        _, marg = lax.fori_loop(a0, a1, scan_tile, (mval0, marg0))
        acc = jnp.where(lane_t == t, marg, acc)
        return (marg, acc)

    prevj0 = jnp.full((RB, 1), -1, jnp.int32)
    acc0 = jnp.zeros((RB, 32), jnp.int32)
    _, acc = lax.fori_loop(0, KNN, extract, (prevj0, acc0))
    idx_ref[...] = acc


def _knn_call(xn, sqr, sqc, batr, batc, t0, t1):
    f = xn.shape[1]
    return pl.pallas_call(
        _knn_body,
        out_shape=jax.ShapeDtypeStruct((N, 32), jnp.int32),
        grid=(NBLK,),
        in_specs=[
            pl.BlockSpec(memory_space=pltpu.SMEM),
            pl.BlockSpec(memory_space=pltpu.SMEM),
            pl.BlockSpec((N, f), lambda i: (0, 0)),
            pl.BlockSpec((RB, 1), lambda i: (i, 0)),
            pl.BlockSpec((1, N), lambda i: (0, 0)),
            pl.BlockSpec((RB, 1), lambda i: (i, 0)),
            pl.BlockSpec((1, N), lambda i: (0, 0)),
        ],
        out_specs=pl.BlockSpec((RB, 32), lambda i: (i, 0)),
        scratch_shapes=[pltpu.VMEM((RB, N), F32)],
    )(t0, t1, xn, sqr, sqc, batr, batc)


def _gather_rows(table, idxf):
    b = idxf.shape[0]
    w = table.shape[1]
    nw = 32
    bpw = b // nw
    ch = 128
    nch = bpw // ch
    mesh = plsc.VectorSubcoreMesh(core_axis_name="c", subcore_axis_name="s")

    @functools.partial(
        pl.kernel, mesh=mesh,
        out_type=jax.ShapeDtypeStruct((b, w), F32),
        scratch_types=[
            pltpu.VMEM((ch,), jnp.int32),
            pltpu.VMEM((ch, w), F32),
            pltpu.SemaphoreType.DMA,
        ],
    )
    def gk(tab_hbm, idx_hbm, out_hbm, idx_v, rows_v, sem):
        wid = lax.axis_index("s") * 2 + lax.axis_index("c")
        base = wid * bpw

        def body(j, _):
            off = base + j * ch
            pltpu.sync_copy(idx_hbm.at[pl.ds(off, ch)], idx_v)
            pltpu.async_copy(tab_hbm.at[idx_v], rows_v, sem).wait()
            pltpu.sync_copy(rows_v, out_hbm.at[pl.ds(off, ch)])
            return 0

        lax.fori_loop(0, nch, body, 0)

    return gk(table, idxf)


PB = 64
EB = PB * KNN
NEB = N // PB


def _edge1_body(tab_ref, g_ref, w_ref, b_ref, y_ref):
    xi = tab_ref[:, :64]
    xir = jnp.broadcast_to(xi[:, None, :], (PB, KNN, 64)).reshape(EB, 64)
    xj = g_ref[:, :64]
    e = jnp.concatenate([xir, xj - xir], axis=1).astype(BF16)
    y_ref[...] = jnp.maximum(
        jnp.dot(e, w_ref[...].astype(BF16), preferred_element_type=F32)
        + b_ref[...], 0.0)


def _edge1_call(tab, g2d, w1tp, b1):
    return pl.pallas_call(
        _edge1_body,
        out_shape=jax.ShapeDtypeStruct((N * KNN, 64), F32),
        grid=(NEB,),
        in_specs=[
            pl.BlockSpec((PB, 128), lambda i: (i, 0)),
            pl.BlockSpec((EB, 128), lambda i: (i, 0)),
            pl.BlockSpec((128, 64), lambda i: (0, 0)),
            pl.BlockSpec((1, 64), lambda i: (0, 0)),
        ],
        out_specs=pl.BlockSpec((EB, 64), lambda i: (i, 0)),
    )(tab, g2d, w1tp, b1)


def _edge2_body(y1_ref, mu_ref, den_ref, g_ref, be_ref, w_ref, b_ref, y_ref):
    x = g_ref[...] * (y1_ref[...] - mu_ref[...]) / den_ref[...] + be_ref[...]
    y_ref[...] = jnp.maximum(
        jnp.dot(x.astype(BF16), w_ref[...].astype(BF16),
                preferred_element_type=F32) + b_ref[...], 0.0)


def _edge2_call(y1, mu1, den1, g1, be1, w2t, b2):
    return pl.pallas_call(
        _edge2_body,
        out_shape=jax.ShapeDtypeStruct((N * KNN, 64), F32),
        grid=(NEB,),
        in_specs=[
            pl.BlockSpec((EB, 64), lambda i: (i, 0)),
            pl.BlockSpec((1, 64), lambda i: (0, 0)),
            pl.BlockSpec((1, 64), lambda i: (0, 0)),
            pl.BlockSpec((1, 64), lambda i: (0, 0)),
            pl.BlockSpec((1, 64), lambda i: (0, 0)),
            pl.BlockSpec((64, 64), lambda i: (0, 0)),
            pl.BlockSpec((1, 64), lambda i: (0, 0)),
        ],
        out_specs=pl.BlockSpec((EB, 64), lambda i: (i, 0)),
    )(y1, mu1, den1, g1, be1, w2t, b2)


def _bnmax_body(y2_ref, mu_ref, den_ref, g_ref, be_ref, x_ref):
    z = (g_ref[...] * (y2_ref[...] - mu_ref[...])) / den_ref[...] + be_ref[...]
    x_ref[...] = jnp.max(z.reshape(PB, KNN, 64), axis=1)


def _bnmax_call(y2, mu2, den2, g2, be2):
    return pl.pallas_call(
        _bnmax_body,
        out_shape=jax.ShapeDtypeStruct((N, 64), F32),
        grid=(NEB,),
        in_specs=[
            pl.BlockSpec((EB, 64), lambda i: (i, 0)),
            pl.BlockSpec((1, 64), lambda i: (0, 0)),
            pl.BlockSpec((1, 64), lambda i: (0, 0)),
            pl.BlockSpec((1, 64), lambda i: (0, 0)),
            pl.BlockSpec((1, 64), lambda i: (0, 0)),
        ],
        out_specs=pl.BlockSpec((PB, 64), lambda i: (i, 0)),
    )(y2, mu2, den2, g2, be2)


def _mlp_body(x_ref, w_ref, b_ref, y_ref, s_ref, q_ref):
    y = jnp.maximum(
        jnp.dot(x_ref[...], w_ref[...], precision=lax.Precision.HIGHEST,
                preferred_element_type=F32) + b_ref[...], 0.0)
    y_ref[...] = y
    s_ref[0] = jnp.sum(y, axis=0, keepdims=True)
    q_ref[0] = jnp.sum(y * y, axis=0, keepdims=True)


def _mlp_call(x, wt, b):
    fin = x.shape[1]
    fout = wt.shape[1]
    nb = N // 512
    return pl.pallas_call(
        _mlp_body,
        out_shape=[jax.ShapeDtypeStruct((N, fout), F32),
                   jax.ShapeDtypeStruct((nb, 1, fout), F32),
                   jax.ShapeDtypeStruct((nb, 1, fout), F32)],
        grid=(nb,),
        in_specs=[
            pl.BlockSpec((512, fin), lambda i: (i, 0)),
            pl.BlockSpec((fin, fout), lambda i: (0, 0)),
            pl.BlockSpec((1, fout), lambda i: (0, 0)),
        ],
        out_specs=[pl.BlockSpec((512, fout), lambda i: (i, 0)),
                   pl.BlockSpec((1, 1, fout), lambda i: (i, 0, 0)),
                   pl.BlockSpec((1, 1, fout), lambda i: (i, 0, 0))],
    )(x, wt, b)


def _final_body(x_ref, w_ref, b_ref, o_ref):
    z = jnp.dot(x_ref[...], w_ref[...], precision=lax.Precision.HIGHEST,
                preferred_element_type=F32) + b_ref[...]
    mx = jnp.max(z, axis=1, keepdims=True)
    ls = jnp.log(jnp.sum(jnp.exp(z - mx), axis=1, keepdims=True)) + mx
    o_ref[...] = z - ls


def _final_call(x, wt, b):
    return pl.pallas_call(
        _final_body,
        out_shape=jax.ShapeDtypeStruct((N, 13), F32),
        grid=(N // 512,),
        in_specs=[
            pl.BlockSpec((512, 128), lambda i: (i, 0)),
            pl.BlockSpec((128, 13), lambda i: (0, 0)),
            pl.BlockSpec((1, 13), lambda i: (0, 0)),
        ],
        out_specs=pl.BlockSpec((512, 13), lambda i: (i, 0)),
    )(x, wt, b)


def _finish_stats(s, q, n, g, be):
    s = s.reshape(-1, s.shape[-1])
    q = q.reshape(-1, q.shape[-1])
    mu = jnp.sum(s, axis=0) / n
    var = jnp.sum(q, axis=0) / n - mu * mu
    a = g * lax.rsqrt(var + EPS)
    c = be - mu * a
    return a, c


def kernel(pos, batch, params):
    batch32 = batch.astype(jnp.int32)
    gids = jnp.arange(NG, dtype=jnp.int32)
    seg_lo = jnp.searchsorted(batch32, gids, side='left').astype(jnp.int32)
    seg_hi = jnp.searchsorted(batch32, gids, side='right').astype(jnp.int32)
    rb = batch32.reshape(NBLK, RB)
    t0 = (seg_lo[rb[:, 0]] // CT).astype(jnp.int32)
    t1 = ((seg_hi[rb[:, -1]] + CT - 1) // CT).astype(jnp.int32)
    batr = batch32.reshape(N, 1)
    batc = batch32.reshape(1, N)

    x = pos
    feats = []
    for ci in range(3):
        (w1, b1, g1, be1), (w2, b2, g2, be2) = params['conv' + str(ci + 1)]
        f = x.shape[1]
        xp = jnp.pad(x, ((0, 0), (0, 8 - f))) if f < 8 else x
        sq = jnp.sum(x * x, axis=1, keepdims=True)
        idx32 = _knn_call(xp, sq, sq.reshape(1, N), batr, batc, t0, t1)
        idxf = idx32[:, :KNN].reshape(-1)
        tab = jnp.pad(x, ((0, 0), (0, 128 - f)))
        g2d = _gather_rows(tab, idxf)
        w1tp = (jnp.zeros((128, 64), F32)
                .at[:f].set(w1[:, :f].T)
                .at[64:64 + f].set(w1[:, f:].T))
        y1 = _edge1_call(tab, g2d, w1tp, b1[None])
        mu1 = jnp.mean(y1, axis=0)
        den1 = jnp.sqrt(jnp.var(y1, axis=0) + EPS)
        y2 = _edge2_call(y1, mu1[None], den1[None], g1[None], be1[None],
                         w2.T, b2[None])
        mu2 = jnp.mean(y2, axis=0)
        den2 = jnp.sqrt(jnp.var(y2, axis=0) + EPS)
        x = _bnmax_call(y2, mu2[None], den2[None], g2[None], be2[None])
        feats.append(x)

    cat = jnp.concatenate(feats, axis=1)
    (wl, bl, gl, bel) = params['lin1'][0]
    y, s, q = _mlp_call(cat, wl.T, bl[None, :])
    a, c = _finish_stats(s, q, float(N), gl, bel)
    (wh, bh, gh, beh) = params['head1'][0]
    y, s, q = _mlp_call(y, wh.T * a[:, None], (bh + wh @ c)[None, :])
    a, c = _finish_stats(s, q, float(N), gh, beh)
    (wh2, bh2, gh2, beh2) = params['head2'][0]
    y, s, q = _mlp_call(y, wh2.T * a[:, None], (bh2 + wh2 @ c)[None, :])
    a, c = _finish_stats(s, q, float(N), gh2, beh2)
    wf, bf = params['final']
    return _final_call(y, wf.T * a[:, None], (bf + wf @ c)[None, :])
